# Initial kernel scaffold; baseline (speedup 1.0000x reference)
import functools

import jax
import jax.numpy as jnp
from jax import lax
from jax.experimental import pallas as pl
from jax.experimental.pallas import tpu as pltpu
from jax.experimental.pallas import tpu_sc as plsc

VOCAB = 1000000
EMBED_DIM = 32
BATCH = 16384
HIST = 50
B = BATCH * HIST

NUM_CORES = 2
NUM_SUBCORES = 16
NW = NUM_CORES * NUM_SUBCORES
B_PER_W = B // NW
CHUNK = 3200
N_ITERS = B_PER_W // CHUNK

_mesh = plsc.VectorSubcoreMesh(core_axis_name="c", subcore_axis_name="s")


@functools.partial(
    pl.kernel,
    out_type=jax.ShapeDtypeStruct((B, EMBED_DIM), jnp.float32),
    mesh=_mesh,
    scratch_types=[
        pltpu.VMEM((CHUNK,), jnp.int32),
        pltpu.VMEM((CHUNK, EMBED_DIM), jnp.float32),
        pltpu.SemaphoreType.DMA,
    ],
)
def _gather_kernel(idx_hbm, table_hbm, out_hbm, idx_v, rows_v, sem):
    wid = lax.axis_index("s") * NUM_CORES + lax.axis_index("c")
    base0 = wid * B_PER_W

    def body(i, carry):
        base = base0 + i * CHUNK
        pltpu.sync_copy(idx_hbm.at[pl.ds(base, CHUNK)], idx_v)
        pltpu.async_copy(table_hbm.at[idx_v], rows_v, sem).wait()
        pltpu.sync_copy(rows_v, out_hbm.at[pl.ds(base, CHUNK)])
        return carry

    lax.fori_loop(0, N_ITERS, body, 0)


def kernel(x, table):
    idx = x.reshape(-1).astype(jnp.int32)
    out = _gather_kernel(idx, table)
    return out.reshape(BATCH, HIST, EMBED_DIM)


# speedup vs baseline: 1.1110x; 1.1110x over previous
import functools

import jax
import jax.numpy as jnp
from jax import lax
from jax.experimental import pallas as pl
from jax.experimental.pallas import tpu as pltpu
from jax.experimental.pallas import tpu_sc as plsc

VOCAB = 1000000
EMBED_DIM = 32
BATCH = 16384
HIST = 50
B = BATCH * HIST

NUM_CORES = 2
NUM_SUBCORES = 16
NW = NUM_CORES * NUM_SUBCORES
B_PER_W = B // NW
CHUNK = 3200
N_ITERS = B_PER_W // CHUNK

_mesh = plsc.VectorSubcoreMesh(core_axis_name="c", subcore_axis_name="s")


@functools.partial(
    pl.kernel,
    out_type=jax.ShapeDtypeStruct((B, EMBED_DIM), jnp.float32),
    mesh=_mesh,
    scratch_types=[
        pltpu.VMEM((CHUNK,), jnp.int32),
        pltpu.VMEM((CHUNK, EMBED_DIM), jnp.float32),
        pltpu.SemaphoreType.DMA,
    ],
    compiler_params=pltpu.CompilerParams(use_tc_tiling_on_sc=False),
)
def _gather_kernel(idx_hbm, table_hbm, out_hbm, idx_v, rows_v, sem):
    wid = lax.axis_index("s") * NUM_CORES + lax.axis_index("c")
    base0 = wid * B_PER_W

    def body(i, carry):
        base = base0 + i * CHUNK
        pltpu.sync_copy(idx_hbm.at[pl.ds(base, CHUNK)], idx_v)
        pltpu.async_copy(table_hbm.at[idx_v], rows_v, sem).wait()
        pltpu.sync_copy(rows_v, out_hbm.at[pl.ds(base, CHUNK)])
        return carry

    lax.fori_loop(0, N_ITERS, body, 0)


def kernel(x, table):
    idx = x.reshape(-1).astype(jnp.int32)
    out = _gather_kernel(idx, table)
    return out.reshape(BATCH, HIST, EMBED_DIM)


# speedup vs baseline: 1.1137x; 1.0024x over previous
import functools

import jax
import jax.numpy as jnp
from jax import lax
from jax.experimental import pallas as pl
from jax.experimental.pallas import tpu as pltpu
from jax.experimental.pallas import tpu_sc as plsc

VOCAB = 1000000
EMBED_DIM = 32
BATCH = 16384
HIST = 50
B = BATCH * HIST

NUM_CORES = 2
NUM_SUBCORES = 16
NW = NUM_CORES * NUM_SUBCORES
B_PER_W = B // NW
CHUNK = 800
NBUF = 4
N_CHUNKS = B_PER_W // CHUNK
N_GROUPS = N_CHUNKS // NBUF

_mesh = plsc.VectorSubcoreMesh(core_axis_name="c", subcore_axis_name="s")


@functools.partial(
    pl.kernel,
    out_type=jax.ShapeDtypeStruct((B, EMBED_DIM), jnp.float32),
    mesh=_mesh,
    scratch_types=[
        pltpu.VMEM((B_PER_W,), jnp.int32),
        [pltpu.VMEM((CHUNK, EMBED_DIM), jnp.float32) for _ in range(NBUF)],
        [pltpu.SemaphoreType.DMA for _ in range(NBUF)],
        [pltpu.SemaphoreType.DMA for _ in range(NBUF)],
    ],
    compiler_params=pltpu.CompilerParams(use_tc_tiling_on_sc=False),
)
def _gather_kernel(idx_hbm, table_hbm, out_hbm, idx_all, rows, sem_g, sem_s):
    wid = lax.axis_index("s") * NUM_CORES + lax.axis_index("c")
    base0 = wid * B_PER_W

    pltpu.sync_copy(idx_hbm.at[pl.ds(base0, B_PER_W)], idx_all)

    def fire_gather(c, b):
        pltpu.async_copy(
            table_hbm.at[idx_all.at[pl.ds(c * CHUNK, CHUNK)]], rows[b], sem_g[b]
        )

    def wait_gather(b):
        pltpu.make_async_copy(
            table_hbm.at[idx_all.at[pl.ds(0, CHUNK)]], rows[b], sem_g[b]
        ).wait()

    def fire_store(c, b):
        pltpu.async_copy(
            rows[b], out_hbm.at[pl.ds(base0 + c * CHUNK, CHUNK)], sem_s[b]
        )

    def wait_store(b):
        pltpu.make_async_copy(
            rows[b], out_hbm.at[pl.ds(base0, CHUNK)], sem_s[b]
        ).wait()


    fire_gather(0, 0)
    fire_gather(1, 1)
    fire_gather(2, 2)
    wait_gather(0)
    fire_store(0, 0)
    fire_gather(3, 3)
    wait_gather(1)
    fire_store(1, 1)
    wait_store(0)
    fire_gather(4, 0)
    wait_gather(2)
    fire_store(2, 2)
    wait_store(1)
    fire_gather(5, 1)
    wait_gather(3)
    fire_store(3, 3)

    def group(g, carry):
        for b in range(NBUF):
            c = g * NBUF + b
            b2 = (b + 2) % NBUF
            wait_store(b2)
            fire_gather(c + 2, b2)
            wait_gather(b)
            fire_store(c, b)
        return carry

    lax.fori_loop(1, N_GROUPS - 1, group, 0)

    cl = (N_GROUPS - 1) * NBUF
    wait_store(2)
    fire_gather(cl + 2, 2)
    wait_gather(0)
    fire_store(cl, 0)
    wait_store(3)
    fire_gather(cl + 3, 3)
    wait_gather(1)
    fire_store(cl + 1, 1)
    wait_store(0)
    wait_gather(2)
    fire_store(cl + 2, 2)
    wait_store(1)
    wait_gather(3)
    fire_store(cl + 3, 3)
    wait_store(2)
    wait_store(3)


def kernel(x, table):
    idx = x.reshape(-1).astype(jnp.int32)
    out = _gather_kernel(idx, table)
    return out.reshape(BATCH, HIST, EMBED_DIM)


# speedup vs baseline: 1.6283x; 1.4621x over previous
import functools

import jax
import jax.numpy as jnp
from jax import lax
from jax.experimental import pallas as pl
from jax.experimental.pallas import tpu as pltpu
from jax.experimental.pallas import tpu_sc as plsc

VOCAB = 1000000
EMBED_DIM = 32
BATCH = 16384
HIST = 50
B = BATCH * HIST

NUM_CORES = 2
NUM_SUBCORES = 16
NW = NUM_CORES * NUM_SUBCORES
JL = 4
B_PER_W = B // NW
OUT_FLAT = BATCH * HIST * EMBED_DIM

_mesh = plsc.VectorSubcoreMesh(core_axis_name="c", subcore_axis_name="s")


@functools.partial(
    pl.kernel,
    out_type=jax.ShapeDtypeStruct((OUT_FLAT,), jnp.float32),
    mesh=_mesh,
    scratch_types=[
        pltpu.VMEM((B_PER_W,), jnp.int32),
        pltpu.VMEM((B_PER_W,), jnp.int32),
        [pltpu.VMEM((128, EMBED_DIM), jnp.float32) for _ in range(2)],
        [pltpu.VMEM((8 * 128 * 4,), jnp.float32) for _ in range(2)],
        [pltpu.SemaphoreType.DMA for _ in range(2)],
        [pltpu.SemaphoreType.DMA for _ in range(2)],
    ],
    compiler_params=pltpu.CompilerParams(
        use_tc_tiling_on_sc=False, needs_layout_passes=False
    ),
)
def _gather_kernel(idx_hbm, table_hbm, out_hbm, xbuf, idx_t, rows, tr,
                   sem_g, sem_s):
    wid = lax.axis_index("s") * NUM_CORES + lax.axis_index("c")

    iota16 = lax.iota(jnp.int32, 16)
    rowids = [iota16 + 16 * m for m in range(8)]
    pre = [(iota16 + 16 * m) * HIST for m in range(8)]

    pltpu.sync_copy(idx_hbm.at[pl.ds(wid * B_PER_W, B_PER_W)], xbuf)

    def idx_body(h, carry):
        for jl in range(JL):
            base = jl * (128 * HIST) + h
            for m in range(8):
                v = plsc.load_gather(xbuf, [pre[m] + base])
                idx_t[pl.ds((jl * HIST + h) * 128 + 16 * m, 16)] = v
        return carry

    lax.fori_loop(0, HIST, idx_body, 0, unroll=False)

    def fire_gather(c, s):
        pltpu.async_copy(
            table_hbm.at[idx_t.at[pl.ds(c * 128, 128)]], rows[s], sem_g[s]
        )

    def wait_gather(s):
        pltpu.make_async_copy(
            table_hbm.at[idx_t.at[pl.ds(0, 128)]], rows[s], sem_g[s]
        ).wait()

    def transpose(s):
        def t_body(d, carry):
            dcol = jnp.full((16,), 0, jnp.int32) + d
            for m in range(8):
                v = plsc.load_gather(rows[s], [rowids[m], dcol])
                tr[s][pl.ds(d * 128 + 16 * m, 16)] = v
            return carry

        lax.fori_loop(0, EMBED_DIM, t_body, 0, unroll=False)

    def fire_stores(jg, h, s):
        for i in range(4):
            pltpu.async_copy(
                tr[s].at[pl.ds(i * 1024, 1024)],
                out_hbm.at[pl.ds((h * 4 + i) * 131072 + jg * 1024, 1024)],
                sem_s[s],
            )

    def wait_stores(s):
        for _ in range(4):
            pltpu.make_async_copy(
                tr[s].at[pl.ds(0, 1024)],
                out_hbm.at[pl.ds(0, 1024)],
                sem_s[s],
            ).wait()

    for jl in range(JL):
        jg = wid * JL + jl
        c0 = jl * HIST
        fire_gather(c0, 0)

        def h_group(hh, carry):
            for par in range(2):
                h = 2 * hh + par
                s = par

                @pl.when(h <= HIST - 2)
                def _():
                    fire_gather(c0 + h + 1, 1 - s)

                wait_gather(s)

                @pl.when(h >= 2)
                def _():
                    wait_stores(s)

                transpose(s)
                fire_stores(jg, h, s)
            return carry

        lax.fori_loop(0, HIST // 2, h_group, 0, unroll=False)
        wait_stores(0)
        wait_stores(1)


def kernel(x, table):
    idx = x.reshape(-1).astype(jnp.int32)
    flat = _gather_kernel(idx, table)
    o = flat.reshape(HIST, 4, 128, 8, 128)
    o = o.transpose(2, 4, 0, 1, 3)
    return o.reshape(BATCH, HIST, EMBED_DIM)
